# baseline (speedup 1.0000x reference)
import functools

import jax
import jax.numpy as jnp
from jax import lax
from jax.experimental import pallas as pl
from jax.experimental.pallas import tpu as pltpu
from jax.experimental.pallas import tpu_sc as plsc

B, N, D = 32, 8192, 64
C = 129
NC, NS = 2, 16

CH1 = N // NS
SUB1 = CH1 // 128

CH3 = 512
NCH3 = N // CH3
SUB3 = CH3 // 128

_mesh = plsc.VectorSubcoreMesh(
    core_axis_name="c", subcore_axis_name="s", num_cores=NC, num_subcores=NS)
_sc_params = pltpu.CompilerParams(use_tc_tiling_on_sc=False)


@functools.partial(
    pl.kernel,
    out_type=(
        jax.ShapeDtypeStruct((B, C, D), jnp.float32),
        jax.ShapeDtypeStruct((B, C, D), jnp.float32),
        jax.ShapeDtypeStruct((B, C, D), jnp.float32),
        jax.ShapeDtypeStruct((B, C, 16), jnp.float32),
    ),
    mesh=_mesh,
    scratch_types=[
        pltpu.VMEM((SUB1, 128), jnp.int32),
        pltpu.VMEM((CH1, D), jnp.float32),
        pltpu.VMEM((CH1, D), jnp.float32),
        pltpu.VMEM((CH1, D), jnp.float32),
        pltpu.VMEM((CH1, 16), jnp.float32),
        pltpu.VMEM_SHARED((C, D), jnp.float32),
        pltpu.VMEM_SHARED((C, D), jnp.float32),
        pltpu.VMEM_SHARED((C, D), jnp.float32),
        pltpu.VMEM_SHARED((C, 16), jnp.float32),
    ],
    compiler_params=_sc_params,
)
def _seg_sums(q_hbm, k_hbm, v_hbm, cl_hbm, zeros_hbm, zeros16_hbm, ones_hbm,
              segq_hbm, segk_hbm, segv_hbm, cnt_hbm,
              idx_v, qb, kb, vb, ones_v, aq, ak, av, ac):
    s = lax.axis_index("s")
    c = lax.axis_index("c")
    pltpu.sync_copy(ones_hbm, ones_v)

    def batch_body(i, carry):
        b = i * NC + c

        @pl.when(s == 0)
        def _zero():
            pltpu.sync_copy(zeros_hbm, aq)
            pltpu.sync_copy(zeros_hbm, ak)
            pltpu.sync_copy(zeros_hbm, av)
            pltpu.sync_copy(zeros16_hbm, ac)

        plsc.subcore_barrier()

        pltpu.sync_copy(cl_hbm.at[b, s], idx_v)
        pltpu.sync_copy(q_hbm.at[b, s], qb)
        pltpu.sync_copy(k_hbm.at[b, s], kb)
        pltpu.sync_copy(v_hbm.at[b, s], vb)
        for j in range(SUB1):
            row = idx_v.at[j]
            sl = pl.ds(j * 128, 128)
            pltpu.sync_copy(qb.at[sl], aq.at[row], add=True)
            pltpu.sync_copy(kb.at[sl], ak.at[row], add=True)
            pltpu.sync_copy(vb.at[sl], av.at[row], add=True)
            pltpu.sync_copy(ones_v.at[sl], ac.at[row], add=True)

        plsc.subcore_barrier()

        @pl.when(s == 0)
        def _writeout():
            pltpu.sync_copy(aq, segq_hbm.at[b])
            pltpu.sync_copy(ak, segk_hbm.at[b])
            pltpu.sync_copy(av, segv_hbm.at[b])
            pltpu.sync_copy(ac, cnt_hbm.at[b])

        return carry

    lax.fori_loop(0, NS, batch_body, 0)


def _attn_body(segq_ref, segk_ref, segv_ref, cnt_ref, v2_ref, acol_ref):
    cnt = jnp.sum(cnt_ref[0], axis=1) * (1.0 / 16.0)
    inv = 1.0 / cnt
    qc = segq_ref[0] * inv[:, None]
    kc = segk_ref[0] * inv[:, None]
    vc = segv_ref[0] * inv[:, None]

    qk = lax.dot_general(qc, kc, (((1,), (1,)), ((), ())),
                         preferred_element_type=jnp.float32)
    a = jax.nn.softmax(qk, axis=-1)
    aw = a * cnt[None, :]
    aw = aw / jnp.sum(aw, axis=-1, keepdims=True)

    v2_ref[0] = jnp.dot(aw, vc, preferred_element_type=jnp.float32)
    col0 = (lax.broadcasted_iota(jnp.int32, (C, C), 1) == 0).astype(jnp.float32)
    acol_ref[0, 0, :] = jnp.sum(aw * col0, axis=1)


def _attn(segq, segk, segv, cnt16):
    return pl.pallas_call(
        _attn_body,
        grid=(B,),
        in_specs=[
            pl.BlockSpec((1, C, D), lambda b: (b, 0, 0)),
            pl.BlockSpec((1, C, D), lambda b: (b, 0, 0)),
            pl.BlockSpec((1, C, D), lambda b: (b, 0, 0)),
            pl.BlockSpec((1, C, 16), lambda b: (b, 0, 0)),
        ],
        out_specs=[
            pl.BlockSpec((1, C, D), lambda b: (b, 0, 0)),
            pl.BlockSpec((1, 1, C), lambda b: (b, 0, 0)),
        ],
        out_shape=[
            jax.ShapeDtypeStruct((B, C, D), jnp.float32),
            jax.ShapeDtypeStruct((B, 1, C), jnp.float32),
        ],
    )(segq, segk, segv, cnt16)


@functools.partial(
    pl.kernel,
    out_type=jax.ShapeDtypeStruct((B, NCH3, CH3, D), jnp.float32),
    mesh=_mesh,
    scratch_types=[
        pltpu.VMEM((SUB3, 128), jnp.int32),
        pltpu.VMEM((CH3, D), jnp.float32),
        pltpu.SemaphoreType.DMA,
    ],
    compiler_params=_sc_params,
)
def _bcast_gather(v2_hbm, gcl_hbm, out_hbm, idx_v, rows, sem):
    b = lax.axis_index("s") * NC + lax.axis_index("c")

    def body(ci, carry):
        pltpu.sync_copy(gcl_hbm.at[b, ci], idx_v)
        for j in range(SUB3):
            pltpu.async_copy(v2_hbm.at[idx_v.at[j]],
                             rows.at[pl.ds(j * 128, 128)], sem).wait()
        pltpu.sync_copy(rows, out_hbm.at[b, ci])
        return carry

    lax.fori_loop(0, NCH3, body, 0)


def kernel(queries, keys, values, clusters):
    q4 = queries.reshape(B, NS, CH1, D)
    k4 = keys.reshape(B, NS, CH1, D)
    v4 = values.reshape(B, NS, CH1, D)
    cl1 = clusters.reshape(B, NS, SUB1, 128)
    zeros = jnp.zeros((C, D), jnp.float32)
    zeros16 = jnp.zeros((C, 16), jnp.float32)
    ones = jnp.ones((CH1, 16), jnp.float32)

    segq, segk, segv, cnt16 = _seg_sums(q4, k4, v4, cl1, zeros, zeros16, ones)
    v2, acol = _attn(segq, segk, segv, cnt16)

    gcl = (clusters + C * jnp.arange(B, dtype=jnp.int32)[:, None])
    gcl = gcl.reshape(B, NCH3, SUB3, 128)
    out4 = _bcast_gather(v2.reshape(B * C, D), gcl)
    return (out4.reshape(B, N, D), acol.reshape(B, C))
